# Initial kernel scaffold; baseline (speedup 1.0000x reference)
import jax
import jax.numpy as jnp
from jax.experimental import pallas as pl


def kernel(query_labels, node_idx, hyperedge_idx, params):
    raise NotImplementedError("write your pallas kernel here")


# speedup vs baseline: 3.4685x; 3.4685x over previous
import functools

import jax
import jax.numpy as jnp
from jax import lax
from jax.experimental import pallas as pl
from jax.experimental.pallas import tpu as pltpu
from jax.experimental.pallas import tpu_sc as plsc

NN = 10000
NH = 20000
NNZ = 320000
D = 128
CW = 16
DW = D + CW
NCLS = 40

PB = 128
NBLK = NNZ // PB
NSC = 2
NT = 16
HALF = NH // NSC
ZSTRIPE = 632
ACC = NT * ZSTRIPE
OSTRIPE = 632
OLAST = HALF - (NT - 1) * OSTRIPE
NNP = ACC
NBLKP = NNP // PB


def _ln(x, g, b):
    mu = jnp.mean(x, axis=-1, keepdims=True)
    var = jnp.mean((x - mu) ** 2, axis=-1, keepdims=True)
    return (x - mu) * lax.rsqrt(var + 1e-5) * g + b


def _mlp2(x, g0, b0, W1, bb1, g1, b1b, W2, bb2):
    x = _ln(x, g0, b0)
    x = jnp.maximum(jnp.dot(x, W1, preferred_element_type=jnp.float32) + bb1, 0.0)
    x = _ln(x, g1, b1b)
    return jnp.dot(x, W2, preferred_element_type=jnp.float32) + bb2


def _full_spec(a):
    return pl.BlockSpec(a.shape, lambda i: tuple(0 for _ in a.shape))


def _row_spec(rb, w):
    return pl.BlockSpec((rb, w), lambda i: (i, 0))


def _widen(x):
    return jnp.concatenate(
        [x, jnp.ones((x.shape[0], CW), jnp.float32)], axis=-1)


def _enc_tc(emb, ps):
    RB = 1000

    def body(emb_ref, g0, b0, W1, bb1, g1, b1b, W2, bb2, out_ref):
        x = _mlp2(emb_ref[...], g0[...], b0[...], W1[...], bb1[...],
                  g1[...], b1b[...], W2[...], bb2[...])
        out_ref[...] = _widen(jnp.maximum(x, 0.0))

    return pl.pallas_call(
        body,
        grid=(NN // RB,),
        in_specs=[_row_spec(RB, D)] + [_full_spec(a) for a in ps],
        out_specs=_row_spec(RB, DW),
        out_shape=jax.ShapeDtypeStruct((NN, DW), jnp.float32),
    )(emb, *ps)


def _mid_tc(acc_e, ps_dec, ps_enc):
    RB = 1000

    def body(a_ref, *refs):
        pd = [r[...] for r in refs[:8]]
        pe = [r[...] for r in refs[8:16]]
        out_ref = refs[16]
        a = a_ref[...]
        cnt = jnp.maximum(a[:, D:D + 1], 1.0)
        x = a[:, :D] / cnt
        x = jnp.maximum(_mlp2(x, *pd), 0.0)
        out_ref[...] = _widen(jnp.maximum(_mlp2(x, *pe), 0.0))

    return pl.pallas_call(
        body,
        grid=(NH // RB,),
        in_specs=[_row_spec(RB, DW)] + [_full_spec(a) for a in ps_dec + ps_enc],
        out_specs=_row_spec(RB, DW),
        out_shape=jax.ShapeDtypeStruct((NH, DW), jnp.float32),
    )(acc_e, *ps_dec, *ps_enc)


def _final_tc(a0, a1, ps_dec, ps_cls):
    RB = 1000

    def body(a0_ref, a1_ref, *refs):
        pd = [r[...] for r in refs[:8]]
        W1, bb1, g, b, W2, bb2 = [r[...] for r in refs[8:14]]
        out_ref = refs[14]
        a = a0_ref[...] + a1_ref[...]
        cnt = jnp.maximum(a[:, D:D + 1], 1.0)
        x = a[:, :D] / cnt
        x = jnp.maximum(_mlp2(x, *pd), 0.0)
        x = jnp.maximum(jnp.dot(x, W1, preferred_element_type=jnp.float32) + bb1, 0.0)
        x = _ln(x, g, b)
        out_ref[...] = jnp.dot(x, W2, preferred_element_type=jnp.float32) + bb2

    return pl.pallas_call(
        body,
        grid=(NN // RB,),
        in_specs=[_row_spec(RB, DW), _row_spec(RB, DW)]
        + [_full_spec(a) for a in ps_dec + ps_cls],
        out_specs=_row_spec(RB, D),
        out_shape=jax.ShapeDtypeStruct((NN, D), jnp.float32),
    )(a0, a1, *ps_dec, *ps_cls)


_MESH = plsc.VectorSubcoreMesh(core_axis_name="c", subcore_axis_name="s")


def _perm_sc(t, qlp):
    @functools.partial(
        pl.kernel,
        out_type=jax.ShapeDtypeStruct((NNP, DW), jnp.float32),
        mesh=_MESH,
        scratch_types=[
            pltpu.VMEM((PB,), jnp.int32),
            pltpu.VMEM((PB, DW), jnp.float32),
            pltpu.SemaphoreType.DMA,
        ],
        compiler_params=pltpu.CompilerParams(use_tc_tiling_on_sc=False),
    )
    def k(t_h, ql_h, out_h, idx_v, rows_v, sem):
        c = lax.axis_index("c")
        s = lax.axis_index("s")
        wid = s * NSC + c

        def blk(i, carry):
            b = wid + i * (NSC * NT)

            @pl.when(b < NBLKP)
            def _():
                off = b * PB
                pltpu.sync_copy(ql_h.at[pl.ds(off, PB)], idx_v)
                pltpu.async_copy(t_h.at[idx_v], rows_v, sem).wait()
                pltpu.sync_copy(rows_v, out_h.at[pl.ds(off, PB)])

            return carry

        lax.fori_loop(0, (NBLKP + NSC * NT - 1) // (NSC * NT), blk, 0)

    return k(t, qlp)


def _v2e_sc(tq, nid, hid, zsum):
    @functools.partial(
        pl.kernel,
        out_type=jax.ShapeDtypeStruct((NH, DW), jnp.float32),
        mesh=_MESH,
        scratch_types=[
            pltpu.VMEM((PB,), jnp.int32),
            pltpu.VMEM((PB,), jnp.int32),
            pltpu.VMEM((PB,), jnp.int32),
            pltpu.VMEM((PB, DW), jnp.float32),
            pltpu.VMEM_SHARED((ACC, DW), jnp.float32),
            pltpu.SemaphoreType.DMA,
        ],
        compiler_params=pltpu.CompilerParams(use_tc_tiling_on_sc=False),
    )
    def k(tq_h, nid_h, hid_h, zs_h, sum_h,
          nid_v, hid_v, dst_v, rows_v, acc_sh, sem):
        c = lax.axis_index("c")
        s = lax.axis_index("s")
        base = c * HALF
        pltpu.sync_copy(zs_h.at[pl.ds(s * ZSTRIPE, ZSTRIPE)],
                        acc_sh.at[pl.ds(s * ZSTRIPE, ZSTRIPE)])
        plsc.subcore_barrier()
        iota = lax.iota(jnp.int32, 16)
        dummy = HALF + (iota & (NT - 1))

        def blk(i, carry):
            b = s + i * NT

            @pl.when(b < NBLK)
            def _():
                off = b * PB
                pltpu.sync_copy(hid_h.at[pl.ds(off, PB)], hid_v)
                pltpu.sync_copy(nid_h.at[pl.ds(off, PB)], nid_v)
                for j in range(PB // 16):
                    sl = pl.ds(j * 16, 16)
                    loc = hid_v[sl] - base
                    okm = jnp.logical_and(loc >= 0, loc < HALF)
                    dst_v[sl] = jnp.where(okm, loc, dummy)
                pltpu.async_copy(tq_h.at[nid_v], rows_v, sem).wait()
                pltpu.sync_copy(rows_v, acc_sh.at[dst_v], add=True)

            return carry

        lax.fori_loop(0, (NBLK + NT - 1) // NT, blk, 0)
        plsc.subcore_barrier()

        @pl.when(s < NT - 1)
        def _():
            pltpu.sync_copy(acc_sh.at[pl.ds(s * OSTRIPE, OSTRIPE)],
                            sum_h.at[pl.ds(base + s * OSTRIPE, OSTRIPE)])

        @pl.when(s == NT - 1)
        def _():
            pltpu.sync_copy(acc_sh.at[pl.ds((NT - 1) * OSTRIPE, OLAST)],
                            sum_h.at[pl.ds(base + (NT - 1) * OSTRIPE, OLAST)])

    return k(tq, nid, hid, zsum)


def _e2v_sc(xe, nid, hid, zsum):
    @functools.partial(
        pl.kernel,
        out_type=jax.ShapeDtypeStruct((NSC, NN, DW), jnp.float32),
        mesh=_MESH,
        scratch_types=[
            pltpu.VMEM((PB,), jnp.int32),
            pltpu.VMEM((PB,), jnp.int32),
            pltpu.VMEM((PB, DW), jnp.float32),
            pltpu.VMEM_SHARED((ACC, DW), jnp.float32),
            pltpu.SemaphoreType.DMA,
        ],
        compiler_params=pltpu.CompilerParams(use_tc_tiling_on_sc=False),
    )
    def k(xe_h, nid_h, hid_h, zs_h, sum_h,
          nid_v, hid_v, rows_v, acc_sh, sem):
        c = lax.axis_index("c")
        s = lax.axis_index("s")
        wid = s * NSC + c
        pltpu.sync_copy(zs_h.at[pl.ds(s * ZSTRIPE, ZSTRIPE)],
                        acc_sh.at[pl.ds(s * ZSTRIPE, ZSTRIPE)])
        plsc.subcore_barrier()

        def blk(i, carry):
            b = wid + i * (NSC * NT)

            @pl.when(b < NBLK)
            def _():
                off = b * PB
                pltpu.sync_copy(hid_h.at[pl.ds(off, PB)], hid_v)
                pltpu.sync_copy(nid_h.at[pl.ds(off, PB)], nid_v)
                pltpu.async_copy(xe_h.at[hid_v], rows_v, sem).wait()
                pltpu.sync_copy(rows_v, acc_sh.at[nid_v], add=True)

            return carry

        lax.fori_loop(0, (NBLK + NSC * NT - 1) // (NSC * NT), blk, 0)
        plsc.subcore_barrier()

        @pl.when(s < NT - 1)
        def _():
            pltpu.sync_copy(acc_sh.at[pl.ds(s * OSTRIPE, OSTRIPE)],
                            sum_h.at[c, pl.ds(s * OSTRIPE, OSTRIPE)])

        @pl.when(s == NT - 1)
        def _():
            pltpu.sync_copy(acc_sh.at[pl.ds((NT - 1) * OSTRIPE, OLAST)],
                            sum_h.at[c, pl.ds((NT - 1) * OSTRIPE, OLAST)])

    return k(xe, nid, hid, zsum)


def _mlp_params(p, prefix):
    return [
        p[prefix + '_ln0_g'].reshape(1, D), p[prefix + '_ln0_b'].reshape(1, D),
        p[prefix + '_W1'], p[prefix + '_b1'].reshape(1, D),
        p[prefix + '_ln1_g'].reshape(1, D), p[prefix + '_ln1_b'].reshape(1, D),
        p[prefix + '_W2'], p[prefix + '_b2'].reshape(1, D),
    ]


def kernel(query_labels, node_idx, hyperedge_idx, params):
    p = params
    ql = query_labels.astype(jnp.int32)
    nid = node_idx.astype(jnp.int32)
    hid = hyperedge_idx.astype(jnp.int32)

    zsum = jnp.zeros((ACC, DW), jnp.float32)

    t = _enc_tc(p['emb'], _mlp_params(p, 'v2e_enc'))
    qlp = jnp.concatenate([ql, jnp.zeros((NNP - NN,), jnp.int32)])
    tq = _perm_sc(t, qlp)
    acc_e = _v2e_sc(tq, nid, hid, zsum)
    xe = _mid_tc(acc_e, _mlp_params(p, 'v2e_dec'), _mlp_params(p, 'e2v_enc'))
    a2 = _e2v_sc(xe, nid, hid, zsum)

    W2p = jnp.zeros((D, D), jnp.float32).at[:, :NCLS].set(p['cls_W2'])
    b2p = jnp.zeros((1, D), jnp.float32).at[:, :NCLS].set(p['cls_b2'])
    ps_cls = [p['cls_W1'], p['cls_b1'].reshape(1, D),
              p['cls_ln_g'].reshape(1, D), p['cls_ln_b'].reshape(1, D),
              W2p, b2p]
    y = _final_tc(a2[0], a2[1], _mlp_params(p, 'e2v_dec'), ps_cls)
    return y[:, :NCLS]


# speedup vs baseline: 5.5917x; 1.6121x over previous
import functools

import jax
import jax.numpy as jnp
from jax import lax
from jax.experimental import pallas as pl
from jax.experimental.pallas import tpu as pltpu
from jax.experimental.pallas import tpu_sc as plsc

NN = 10000
NH = 20000
NNZ = 320000
D = 128
CW = 16
DW = D + CW
NCLS = 40

PB = 128
NBLK = NNZ // PB
SUB = 2
BPB = SUB * PB
NBLK2 = NNZ // BPB
NSC = 2
NT = 16
HALF = NH // NSC
ZSTRIPE = 632
ACC = NT * ZSTRIPE
OSTRIPE = 632
OLAST = HALF - (NT - 1) * OSTRIPE
NNP = ACC
NBLKP = NNP // PB


def _ln(x, g, b):
    mu = jnp.mean(x, axis=-1, keepdims=True)
    var = jnp.mean((x - mu) ** 2, axis=-1, keepdims=True)
    return (x - mu) * lax.rsqrt(var + 1e-5) * g + b


def _mlp2(x, g0, b0, W1, bb1, g1, b1b, W2, bb2):
    x = _ln(x, g0, b0)
    x = jnp.maximum(jnp.dot(x, W1, preferred_element_type=jnp.float32) + bb1, 0.0)
    x = _ln(x, g1, b1b)
    return jnp.dot(x, W2, preferred_element_type=jnp.float32) + bb2


def _full_spec(a):
    return pl.BlockSpec(a.shape, lambda i: tuple(0 for _ in a.shape))


def _row_spec(rb, w):
    return pl.BlockSpec((rb, w), lambda i: (i, 0))


def _widen(x):
    return jnp.concatenate(
        [x, jnp.ones((x.shape[0], CW), jnp.float32)], axis=-1)


def _enc_tc(emb, ps):
    RB = 1000

    def body(emb_ref, g0, b0, W1, bb1, g1, b1b, W2, bb2, out_ref):
        x = _mlp2(emb_ref[...], g0[...], b0[...], W1[...], bb1[...],
                  g1[...], b1b[...], W2[...], bb2[...])
        out_ref[...] = _widen(jnp.maximum(x, 0.0))

    return pl.pallas_call(
        body,
        grid=(NN // RB,),
        in_specs=[_row_spec(RB, D)] + [_full_spec(a) for a in ps],
        out_specs=_row_spec(RB, DW),
        out_shape=jax.ShapeDtypeStruct((NN, DW), jnp.float32),
    )(emb, *ps)


def _mid_tc(acc_e, ps_dec, ps_enc):
    RB = 1000

    def body(a_ref, *refs):
        pd = [r[...] for r in refs[:8]]
        pe = [r[...] for r in refs[8:16]]
        out_ref = refs[16]
        a = a_ref[...]
        cnt = jnp.maximum(a[:, D:D + 1], 1.0)
        x = a[:, :D] / cnt
        x = jnp.maximum(_mlp2(x, *pd), 0.0)
        out_ref[...] = _widen(jnp.maximum(_mlp2(x, *pe), 0.0))

    return pl.pallas_call(
        body,
        grid=(NH // RB,),
        in_specs=[_row_spec(RB, DW)] + [_full_spec(a) for a in ps_dec + ps_enc],
        out_specs=_row_spec(RB, DW),
        out_shape=jax.ShapeDtypeStruct((NH, DW), jnp.float32),
    )(acc_e, *ps_dec, *ps_enc)


def _final_tc(a0, a1, ps_dec, ps_cls):
    RB = 1000

    def body(a0_ref, a1_ref, *refs):
        pd = [r[...] for r in refs[:8]]
        W1, bb1, g, b, W2, bb2 = [r[...] for r in refs[8:14]]
        out_ref = refs[14]
        a = a0_ref[...] + a1_ref[...]
        cnt = jnp.maximum(a[:, D:D + 1], 1.0)
        x = a[:, :D] / cnt
        x = jnp.maximum(_mlp2(x, *pd), 0.0)
        x = jnp.maximum(jnp.dot(x, W1, preferred_element_type=jnp.float32) + bb1, 0.0)
        x = _ln(x, g, b)
        out_ref[...] = jnp.dot(x, W2, preferred_element_type=jnp.float32) + bb2

    return pl.pallas_call(
        body,
        grid=(NN // RB,),
        in_specs=[_row_spec(RB, DW), _row_spec(RB, DW)]
        + [_full_spec(a) for a in ps_dec + ps_cls],
        out_specs=_row_spec(RB, D),
        out_shape=jax.ShapeDtypeStruct((NN, D), jnp.float32),
    )(a0, a1, *ps_dec, *ps_cls)


_MESH = plsc.VectorSubcoreMesh(core_axis_name="c", subcore_axis_name="s")


def _perm_sc(t, qlp):
    @functools.partial(
        pl.kernel,
        out_type=jax.ShapeDtypeStruct((NNP, DW), jnp.float32),
        mesh=_MESH,
        scratch_types=[
            pltpu.VMEM((PB,), jnp.int32),
            pltpu.VMEM((PB, DW), jnp.float32),
            pltpu.SemaphoreType.DMA,
        ],
        compiler_params=pltpu.CompilerParams(use_tc_tiling_on_sc=False),
    )
    def k(t_h, ql_h, out_h, idx_v, rows_v, sem):
        c = lax.axis_index("c")
        s = lax.axis_index("s")
        wid = s * NSC + c

        def blk(i, carry):
            b = wid + i * (NSC * NT)

            @pl.when(b < NBLKP)
            def _():
                off = b * PB
                pltpu.sync_copy(ql_h.at[pl.ds(off, PB)], idx_v)
                pltpu.async_copy(t_h.at[idx_v], rows_v, sem).wait()
                pltpu.sync_copy(rows_v, out_h.at[pl.ds(off, PB)])

            return carry

        lax.fori_loop(0, (NBLKP + NSC * NT - 1) // (NSC * NT), blk, 0)

    return k(t, qlp)


def _v2e_sc(tq, pair, zsum):
    @functools.partial(
        pl.kernel,
        out_type=jax.ShapeDtypeStruct((NH, DW), jnp.float32),
        mesh=_MESH,
        scratch_types=[
            pltpu.VMEM((2, SUB, PB), jnp.int32),
            pltpu.VMEM((SUB, PB), jnp.int32),
            pltpu.VMEM((BPB, DW), jnp.float32),
            pltpu.VMEM_SHARED((ACC, DW), jnp.float32),
            pltpu.SemaphoreType.DMA,
            pltpu.SemaphoreType.DMA,
        ],
        compiler_params=pltpu.CompilerParams(use_tc_tiling_on_sc=False),
    )
    def k(tq_h, pair_h, zs_h, sum_h,
          idx2_v, dst_v, rows_v, acc_sh, gsem, ssem):
        c = lax.axis_index("c")
        s = lax.axis_index("s")
        base = c * HALF
        pltpu.sync_copy(zs_h.at[pl.ds(s * ZSTRIPE, ZSTRIPE)],
                        acc_sh.at[pl.ds(s * ZSTRIPE, ZSTRIPE)])
        plsc.subcore_barrier()
        iota = lax.iota(jnp.int32, 16)
        dummy = HALF + (iota & (NT - 1))

        def drain():
            for g in range(SUB):
                pltpu.make_async_copy(rows_v.at[pl.ds(g * PB, PB)],
                                      acc_sh.at[dst_v.at[g]], ssem).wait()

        def blk(i, carry):
            b = s + i * NT

            def live(carry):
                @pl.when(carry > 0)
                def _():
                    drain()
                pltpu.sync_copy(pair_h.at[b], idx2_v)
                lo = idx2_v[1, 0, pl.ds(0, 16)][0]
                hi = idx2_v[1, SUB - 1, pl.ds(PB - 16, 16)][15]
                own = jnp.logical_and(lo < base + HALF, hi >= base)

                def do_own(_):
                    for g in range(SUB):
                        for j in range(PB // 16):
                            sl = pl.ds(j * 16, 16)
                            loc = idx2_v[1, g, sl] - base
                            okm = jnp.logical_and(loc >= 0, loc < HALF)
                            dst_v[g, sl] = jnp.where(okm, loc, dummy)
                    ds_ = [pltpu.async_copy(tq_h.at[idx2_v.at[0, g]],
                                            rows_v.at[pl.ds(g * PB, PB)], gsem)
                           for g in range(SUB)]
                    for d in ds_:
                        d.wait()
                    for g in range(SUB):
                        pltpu.async_copy(rows_v.at[pl.ds(g * PB, PB)],
                                         acc_sh.at[dst_v.at[g]], ssem, add=True)
                    return SUB

                return lax.cond(own, do_own, lambda _: 0, 0)

            return lax.cond(b < NBLK2, live, lambda cc: cc, carry)

        carry = lax.fori_loop(0, (NBLK2 + NT - 1) // NT, blk, 0)

        @pl.when(carry > 0)
        def _():
            drain()

        plsc.subcore_barrier()

        @pl.when(s < NT - 1)
        def _():
            pltpu.sync_copy(acc_sh.at[pl.ds(s * OSTRIPE, OSTRIPE)],
                            sum_h.at[pl.ds(base + s * OSTRIPE, OSTRIPE)])

        @pl.when(s == NT - 1)
        def _():
            pltpu.sync_copy(acc_sh.at[pl.ds((NT - 1) * OSTRIPE, OLAST)],
                            sum_h.at[pl.ds(base + (NT - 1) * OSTRIPE, OLAST)])

    return k(tq, pair, zsum)


def _e2v_sc(xe, pair, zsum):
    @functools.partial(
        pl.kernel,
        out_type=jax.ShapeDtypeStruct((NSC, NN, DW), jnp.float32),
        mesh=_MESH,
        scratch_types=[
            pltpu.VMEM((2, SUB, PB), jnp.int32),
            pltpu.VMEM((BPB, DW), jnp.float32),
            pltpu.VMEM_SHARED((ACC, DW), jnp.float32),
            pltpu.SemaphoreType.DMA,
            pltpu.SemaphoreType.DMA,
        ],
        compiler_params=pltpu.CompilerParams(use_tc_tiling_on_sc=False),
    )
    def k(xe_h, pair_h, zs_h, sum_h,
          idx2_v, rows_v, acc_sh, gsem, ssem):
        c = lax.axis_index("c")
        s = lax.axis_index("s")
        wid = s * NSC + c
        pltpu.sync_copy(zs_h.at[pl.ds(s * ZSTRIPE, ZSTRIPE)],
                        acc_sh.at[pl.ds(s * ZSTRIPE, ZSTRIPE)])
        plsc.subcore_barrier()

        def drain():
            for g in range(SUB):
                pltpu.make_async_copy(rows_v.at[pl.ds(g * PB, PB)],
                                      acc_sh.at[idx2_v.at[0, g]], ssem).wait()

        def blk(i, carry):
            b = wid + i * (NSC * NT)

            def live(carry):
                @pl.when(carry > 0)
                def _():
                    drain()
                pltpu.sync_copy(pair_h.at[b], idx2_v)
                ds_ = [pltpu.async_copy(xe_h.at[idx2_v.at[1, g]],
                                        rows_v.at[pl.ds(g * PB, PB)], gsem)
                       for g in range(SUB)]
                for d in ds_:
                    d.wait()
                for g in range(SUB):
                    pltpu.async_copy(rows_v.at[pl.ds(g * PB, PB)],
                                     acc_sh.at[idx2_v.at[0, g]], ssem, add=True)
                return SUB

            return lax.cond(b < NBLK2, live, lambda cc: cc, carry)

        carry = lax.fori_loop(0, (NBLK2 + NSC * NT - 1) // (NSC * NT), blk, 0)

        @pl.when(carry > 0)
        def _():
            drain()

        plsc.subcore_barrier()

        @pl.when(s < NT - 1)
        def _():
            pltpu.sync_copy(acc_sh.at[pl.ds(s * OSTRIPE, OSTRIPE)],
                            sum_h.at[c, pl.ds(s * OSTRIPE, OSTRIPE)])

        @pl.when(s == NT - 1)
        def _():
            pltpu.sync_copy(acc_sh.at[pl.ds((NT - 1) * OSTRIPE, OLAST)],
                            sum_h.at[c, pl.ds((NT - 1) * OSTRIPE, OLAST)])

    return k(xe, pair, zsum)


def _mlp_params(p, prefix):
    return [
        p[prefix + '_ln0_g'].reshape(1, D), p[prefix + '_ln0_b'].reshape(1, D),
        p[prefix + '_W1'], p[prefix + '_b1'].reshape(1, D),
        p[prefix + '_ln1_g'].reshape(1, D), p[prefix + '_ln1_b'].reshape(1, D),
        p[prefix + '_W2'], p[prefix + '_b2'].reshape(1, D),
    ]


def kernel(query_labels, node_idx, hyperedge_idx, params):
    p = params
    ql = query_labels.astype(jnp.int32)
    nid = node_idx.astype(jnp.int32)
    hid = hyperedge_idx.astype(jnp.int32)

    zsum = jnp.zeros((ACC, DW), jnp.float32)
    pair = jnp.stack([nid.reshape(NBLK2, SUB, PB),
                      hid.reshape(NBLK2, SUB, PB)], axis=1)

    t = _enc_tc(p['emb'], _mlp_params(p, 'v2e_enc'))
    qlp = jnp.concatenate([ql, jnp.zeros((NNP - NN,), jnp.int32)])
    tq = _perm_sc(t, qlp)
    acc_e = _v2e_sc(tq, pair, zsum)
    xe = _mid_tc(acc_e, _mlp_params(p, 'v2e_dec'), _mlp_params(p, 'e2v_enc'))
    a2 = _e2v_sc(xe, pair, zsum)

    W2p = jnp.zeros((D, D), jnp.float32).at[:, :NCLS].set(p['cls_W2'])
    b2p = jnp.zeros((1, D), jnp.float32).at[:, :NCLS].set(p['cls_b2'])
    ps_cls = [p['cls_W1'], p['cls_b1'].reshape(1, D),
              p['cls_ln_g'].reshape(1, D), p['cls_ln_b'].reshape(1, D),
              W2p, b2p]
    y = _final_tc(a2[0], a2[1], _mlp_params(p, 'e2v_dec'), ps_cls)
    return y[:, :NCLS]


# speedup vs baseline: 5.9885x; 1.0710x over previous
import functools

import jax
import jax.numpy as jnp
from jax import lax
from jax.experimental import pallas as pl
from jax.experimental.pallas import tpu as pltpu
from jax.experimental.pallas import tpu_sc as plsc

NN = 10000
NH = 20000
NNZ = 320000
D = 128
CW = 16
DW = D + CW
NCLS = 40

PB = 128
NBLK = NNZ // PB
SUB = 1
BPB = SUB * PB
NBLK2 = NNZ // BPB
NSC = 2
NT = 16
HALF = NH // NSC
ZSTRIPE = 632
ACC = NT * ZSTRIPE
OSTRIPE = 632
OLAST = HALF - (NT - 1) * OSTRIPE
NNP = ACC
NBLKP = NNP // PB


def _ln(x, g, b):
    mu = jnp.mean(x, axis=-1, keepdims=True)
    var = jnp.mean((x - mu) ** 2, axis=-1, keepdims=True)
    return (x - mu) * lax.rsqrt(var + 1e-5) * g + b


def _mlp2(x, g0, b0, W1, bb1, g1, b1b, W2, bb2):
    x = _ln(x, g0, b0)
    x = jnp.maximum(jnp.dot(x, W1, preferred_element_type=jnp.float32) + bb1, 0.0)
    x = _ln(x, g1, b1b)
    return jnp.dot(x, W2, preferred_element_type=jnp.float32) + bb2


def _full_spec(a):
    return pl.BlockSpec(a.shape, lambda i: tuple(0 for _ in a.shape))


def _row_spec(rb, w):
    return pl.BlockSpec((rb, w), lambda i: (i, 0))


def _widen(x):
    return jnp.concatenate(
        [x, jnp.ones((x.shape[0], CW), jnp.float32)], axis=-1)


def _enc_tc(emb, ps):
    RB = 1000

    def body(emb_ref, g0, b0, W1, bb1, g1, b1b, W2, bb2, out_ref):
        x = _mlp2(emb_ref[...], g0[...], b0[...], W1[...], bb1[...],
                  g1[...], b1b[...], W2[...], bb2[...])
        out_ref[...] = _widen(jnp.maximum(x, 0.0))

    return pl.pallas_call(
        body,
        grid=(NN // RB,),
        in_specs=[_row_spec(RB, D)] + [_full_spec(a) for a in ps],
        out_specs=_row_spec(RB, DW),
        out_shape=jax.ShapeDtypeStruct((NN, DW), jnp.float32),
    )(emb, *ps)


def _mid_tc(acc_e, ps_dec, ps_enc):
    RB = 1000

    def body(a_ref, *refs):
        pd = [r[...] for r in refs[:8]]
        pe = [r[...] for r in refs[8:16]]
        out_ref = refs[16]
        a = a_ref[...]
        cnt = jnp.maximum(a[:, D:D + 1], 1.0)
        x = a[:, :D] / cnt
        x = jnp.maximum(_mlp2(x, *pd), 0.0)
        out_ref[...] = _widen(jnp.maximum(_mlp2(x, *pe), 0.0))

    return pl.pallas_call(
        body,
        grid=(NH // RB,),
        in_specs=[_row_spec(RB, DW)] + [_full_spec(a) for a in ps_dec + ps_enc],
        out_specs=_row_spec(RB, DW),
        out_shape=jax.ShapeDtypeStruct((NH, DW), jnp.float32),
    )(acc_e, *ps_dec, *ps_enc)


def _final_tc(a0, a1, ps_dec, ps_cls):
    RB = 1000

    def body(a0_ref, a1_ref, *refs):
        pd = [r[...] for r in refs[:8]]
        W1, bb1, g, b, W2, bb2 = [r[...] for r in refs[8:14]]
        out_ref = refs[14]
        a = a0_ref[...] + a1_ref[...]
        cnt = jnp.maximum(a[:, D:D + 1], 1.0)
        x = a[:, :D] / cnt
        x = jnp.maximum(_mlp2(x, *pd), 0.0)
        x = jnp.maximum(jnp.dot(x, W1, preferred_element_type=jnp.float32) + bb1, 0.0)
        x = _ln(x, g, b)
        out_ref[...] = jnp.dot(x, W2, preferred_element_type=jnp.float32) + bb2

    return pl.pallas_call(
        body,
        grid=(NN // RB,),
        in_specs=[_row_spec(RB, DW), _row_spec(RB, DW)]
        + [_full_spec(a) for a in ps_dec + ps_cls],
        out_specs=_row_spec(RB, D),
        out_shape=jax.ShapeDtypeStruct((NN, D), jnp.float32),
    )(a0, a1, *ps_dec, *ps_cls)


_MESH = plsc.VectorSubcoreMesh(core_axis_name="c", subcore_axis_name="s")


def _perm_sc(t, qlp):
    @functools.partial(
        pl.kernel,
        out_type=jax.ShapeDtypeStruct((NNP, DW), jnp.float32),
        mesh=_MESH,
        scratch_types=[
            pltpu.VMEM((PB,), jnp.int32),
            pltpu.VMEM((PB, DW), jnp.float32),
            pltpu.SemaphoreType.DMA,
        ],
        compiler_params=pltpu.CompilerParams(use_tc_tiling_on_sc=False),
    )
    def k(t_h, ql_h, out_h, idx_v, rows_v, sem):
        c = lax.axis_index("c")
        s = lax.axis_index("s")
        wid = s * NSC + c

        def blk(i, carry):
            b = wid + i * (NSC * NT)

            @pl.when(b < NBLKP)
            def _():
                off = b * PB
                pltpu.sync_copy(ql_h.at[pl.ds(off, PB)], idx_v)
                pltpu.async_copy(t_h.at[idx_v], rows_v, sem).wait()
                pltpu.sync_copy(rows_v, out_h.at[pl.ds(off, PB)])

            return carry

        lax.fori_loop(0, (NBLKP + NSC * NT - 1) // (NSC * NT), blk, 0)

    return k(t, qlp)


def _gather_scatter_sc(tab, pair, zsum, v2e):
    if v2e:
        out_type = jax.ShapeDtypeStruct((NH, DW), jnp.float32)
        stride = NT
        kmax = (NBLK2 + NT - 1) // NT
    else:
        out_type = jax.ShapeDtypeStruct((NSC, NN, DW), jnp.float32)
        stride = NSC * NT
        kmax = (NBLK2 + NSC * NT - 1) // (NSC * NT)
    gi = 0 if v2e else 1

    @functools.partial(
        pl.kernel,
        out_type=out_type,
        mesh=_MESH,
        scratch_types=[
            pltpu.VMEM((2, 2, SUB, PB), jnp.int32),
            pltpu.VMEM((2, SUB, PB), jnp.int32),
            pltpu.VMEM((2, BPB, DW), jnp.float32),
            pltpu.VMEM_SHARED((ACC, DW), jnp.float32),
            pltpu.SemaphoreType.DMA,
            pltpu.SemaphoreType.DMA,
            pltpu.SemaphoreType.DMA,
            pltpu.SemaphoreType.DMA,
            pltpu.SemaphoreType.DMA,
        ],
        compiler_params=pltpu.CompilerParams(use_tc_tiling_on_sc=False),
    )
    def k(tab_h, pair_h, zs_h, sum_h,
          idx2_v, dst_v, rows_v, acc_sh, isem0, isem1, gsem, ssem0, ssem1):
        c = lax.axis_index("c")
        s = lax.axis_index("s")
        base = c * HALF if v2e else 0
        w0 = s if v2e else s * NSC + c
        isems = (isem0, isem1)
        ssems = (ssem0, ssem1)
        pltpu.sync_copy(zs_h.at[pl.ds(s * ZSTRIPE, ZSTRIPE)],
                        acc_sh.at[pl.ds(s * ZSTRIPE, ZSTRIPE)])
        plsc.subcore_barrier()
        iota = lax.iota(jnp.int32, 16)
        dummy = HALF + (iota & (NT - 1))

        def drain(p):
            for g in range(SUB):
                pltpu.make_async_copy(rows_v.at[p, pl.ds(g * PB, PB)],
                                      acc_sh.at[dst_v.at[p, g]],
                                      ssems[p]).wait()

        def prefetch(b, p):
            @pl.when(b < NBLK2)
            def _():
                pltpu.async_copy(pair_h.at[b], idx2_v.at[p], isems[p])

        def substep(kk, p, cp):
            b = w0 + kk * stride

            def do_own(cp):
                @pl.when(cp > 0)
                def _():
                    drain(p)
                for g in range(SUB):
                    for j in range(PB // 16):
                        sl = pl.ds(j * 16, 16)
                        if v2e:
                            loc = idx2_v[p, 1, g, sl] - base
                            okm = jnp.logical_and(loc >= 0, loc < HALF)
                            dst_v[p, g, sl] = jnp.where(okm, loc, dummy)
                        else:
                            dst_v[p, g, sl] = idx2_v[p, 0, g, sl]
                ds_ = [pltpu.async_copy(tab_h.at[idx2_v.at[p, gi, g]],
                                        rows_v.at[p, pl.ds(g * PB, PB)], gsem)
                       for g in range(SUB)]
                for d in ds_:
                    d.wait()
                for g in range(SUB):
                    pltpu.async_copy(rows_v.at[p, pl.ds(g * PB, PB)],
                                     acc_sh.at[dst_v.at[p, g]],
                                     ssems[p], add=True)
                return SUB

            def live(cp):
                pltpu.make_async_copy(pair_h.at[b], idx2_v.at[p],
                                      isems[p]).wait()
                prefetch(b + stride, p ^ 1)
                if v2e:
                    lo = idx2_v[p, 1, 0, pl.ds(0, 16)][0]
                    hi = idx2_v[p, 1, SUB - 1, pl.ds(PB - 16, 16)][15]
                    own = jnp.logical_and(lo < base + HALF, hi >= base)
                    return lax.cond(own, do_own, lambda cc: cc, cp)
                return do_own(cp)

            return lax.cond(b < NBLK2, live, lambda cc: cc, cp)

        prefetch(w0, 0)

        def outer(k2, carry):
            c0, c1 = carry
            c0 = substep(2 * k2, 0, c0)
            c1 = substep(2 * k2 + 1, 1, c1)
            return (c0, c1)

        c0, c1 = lax.fori_loop(0, (kmax + 1) // 2, outer, (0, 0))

        @pl.when(c0 > 0)
        def _():
            drain(0)

        @pl.when(c1 > 0)
        def _():
            drain(1)

        plsc.subcore_barrier()

        def out_at(off, n):
            if v2e:
                return sum_h.at[pl.ds(base + off, n)]
            return sum_h.at[c, pl.ds(off, n)]

        @pl.when(s < NT - 1)
        def _():
            pltpu.sync_copy(acc_sh.at[pl.ds(s * OSTRIPE, OSTRIPE)],
                            out_at(s * OSTRIPE, OSTRIPE))

        @pl.when(s == NT - 1)
        def _():
            pltpu.sync_copy(acc_sh.at[pl.ds((NT - 1) * OSTRIPE, OLAST)],
                            out_at((NT - 1) * OSTRIPE, OLAST))

    return k(tab, pair, zsum)


def _v2e_sc(tq, pair, zsum):
    return _gather_scatter_sc(tq, pair, zsum, v2e=True)


def _e2v_sc(xe, pair, zsum):
    return _gather_scatter_sc(xe, pair, zsum, v2e=False)


def _mlp_params(p, prefix):
    return [
        p[prefix + '_ln0_g'].reshape(1, D), p[prefix + '_ln0_b'].reshape(1, D),
        p[prefix + '_W1'], p[prefix + '_b1'].reshape(1, D),
        p[prefix + '_ln1_g'].reshape(1, D), p[prefix + '_ln1_b'].reshape(1, D),
        p[prefix + '_W2'], p[prefix + '_b2'].reshape(1, D),
    ]


def kernel(query_labels, node_idx, hyperedge_idx, params):
    p = params
    ql = query_labels.astype(jnp.int32)
    nid = node_idx.astype(jnp.int32)
    hid = hyperedge_idx.astype(jnp.int32)

    zsum = jnp.zeros((ACC, DW), jnp.float32)
    pair = jnp.stack([nid.reshape(NBLK2, SUB, PB),
                      hid.reshape(NBLK2, SUB, PB)], axis=1)

    t = _enc_tc(p['emb'], _mlp_params(p, 'v2e_enc'))
    qlp = jnp.concatenate([ql, jnp.zeros((NNP - NN,), jnp.int32)])
    tq = _perm_sc(t, qlp)
    acc_e = _v2e_sc(tq, pair, zsum)
    xe = _mid_tc(acc_e, _mlp_params(p, 'v2e_dec'), _mlp_params(p, 'e2v_enc'))
    a2 = _e2v_sc(xe, pair, zsum)

    W2p = jnp.zeros((D, D), jnp.float32).at[:, :NCLS].set(p['cls_W2'])
    b2p = jnp.zeros((1, D), jnp.float32).at[:, :NCLS].set(p['cls_b2'])
    ps_cls = [p['cls_W1'], p['cls_b1'].reshape(1, D),
              p['cls_ln_g'].reshape(1, D), p['cls_ln_b'].reshape(1, D),
              W2p, b2p]
    y = _final_tc(a2[0], a2[1], _mlp_params(p, 'e2v_dec'), ps_cls)
    return y[:, :NCLS]


# speedup vs baseline: 6.8086x; 1.1370x over previous
import functools

import jax
import jax.numpy as jnp
from jax import lax
from jax.experimental import pallas as pl
from jax.experimental.pallas import tpu as pltpu
from jax.experimental.pallas import tpu_sc as plsc

NN = 10000
NH = 20000
NNZ = 320000
D = 128
CW = 16
NCLS = 40

PB = 128
SUB = 1
BPB = SUB * PB
NBLK2 = NNZ // BPB
NSC = 2
NT = 16
HALF = NH // NSC
ZSTRIPE = 632
ACC = NT * ZSTRIPE
OSTRIPE = 632
OLAST = HALF - (NT - 1) * OSTRIPE
NNP = ACC
NBLKP = NNP // PB


def _ln(x, g, b):
    mu = jnp.mean(x, axis=-1, keepdims=True)
    var = jnp.mean((x - mu) ** 2, axis=-1, keepdims=True)
    return (x - mu) * lax.rsqrt(var + 1e-5) * g + b


def _mlp2(x, g0, b0, W1, bb1, g1, b1b, W2, bb2):
    x = _ln(x, g0, b0)
    x = jnp.maximum(jnp.dot(x, W1, preferred_element_type=jnp.float32) + bb1, 0.0)
    x = _ln(x, g1, b1b)
    return jnp.dot(x, W2, preferred_element_type=jnp.float32) + bb2


def _full_spec(a):
    return pl.BlockSpec(a.shape, lambda i: tuple(0 for _ in a.shape))


def _row_spec(rb, w):
    return pl.BlockSpec((rb, w), lambda i: (i, 0))


def _enc_tc(emb, ps):
    RB = 1000

    def body(emb_ref, g0, b0, W1, bb1, g1, b1b, W2, bb2, out_ref):
        x = _mlp2(emb_ref[...], g0[...], b0[...], W1[...], bb1[...],
                  g1[...], b1b[...], W2[...], bb2[...])
        out_ref[...] = jnp.maximum(x, 0.0)

    return pl.pallas_call(
        body,
        grid=(NN // RB,),
        in_specs=[_row_spec(RB, D)] + [_full_spec(a) for a in ps],
        out_specs=_row_spec(RB, D),
        out_shape=jax.ShapeDtypeStruct((NN, D), jnp.float32),
    )(emb, *ps)


def _mid_tc(sums, cnts, ps_dec, ps_enc):
    RB = 1000

    def body(s_ref, c_ref, *refs):
        pd = [r[...] for r in refs[:8]]
        pe = [r[...] for r in refs[8:16]]
        out_ref = refs[16]
        cnt = jnp.maximum(c_ref[...][:, :1], 1.0)
        x = s_ref[...] / cnt
        x = jnp.maximum(_mlp2(x, *pd), 0.0)
        out_ref[...] = jnp.maximum(_mlp2(x, *pe), 0.0)

    return pl.pallas_call(
        body,
        grid=(NH // RB,),
        in_specs=[_row_spec(RB, D), _row_spec(RB, CW)]
        + [_full_spec(a) for a in ps_dec + ps_enc],
        out_specs=_row_spec(RB, D),
        out_shape=jax.ShapeDtypeStruct((NH, D), jnp.float32),
    )(sums, cnts, *ps_dec, *ps_enc)


def _final_tc(s0, s1, c0, c1, ps_dec, ps_cls):
    RB = 1000

    def body(s0_ref, s1_ref, c0_ref, c1_ref, *refs):
        pd = [r[...] for r in refs[:8]]
        W1, bb1, g, b, W2, bb2 = [r[...] for r in refs[8:14]]
        out_ref = refs[14]
        cnt = jnp.maximum(c0_ref[...][:, :1] + c1_ref[...][:, :1], 1.0)
        x = (s0_ref[...] + s1_ref[...]) / cnt
        x = jnp.maximum(_mlp2(x, *pd), 0.0)
        x = jnp.maximum(jnp.dot(x, W1, preferred_element_type=jnp.float32) + bb1, 0.0)
        x = _ln(x, g, b)
        out_ref[...] = jnp.dot(x, W2, preferred_element_type=jnp.float32) + bb2

    return pl.pallas_call(
        body,
        grid=(NN // RB,),
        in_specs=[_row_spec(RB, D), _row_spec(RB, D),
                  _row_spec(RB, CW), _row_spec(RB, CW)]
        + [_full_spec(a) for a in ps_dec + ps_cls],
        out_specs=_row_spec(RB, D),
        out_shape=jax.ShapeDtypeStruct((NN, D), jnp.float32),
    )(s0, s1, c0, c1, *ps_dec, *ps_cls)


_MESH = plsc.VectorSubcoreMesh(core_axis_name="c", subcore_axis_name="s")


def _perm_sc(t, qlp):
    @functools.partial(
        pl.kernel,
        out_type=jax.ShapeDtypeStruct((NNP, D), jnp.float32),
        mesh=_MESH,
        scratch_types=[
            pltpu.VMEM((PB,), jnp.int32),
            pltpu.VMEM((PB, D), jnp.float32),
            pltpu.SemaphoreType.DMA,
        ],
        compiler_params=pltpu.CompilerParams(use_tc_tiling_on_sc=False),
    )
    def k(t_h, ql_h, out_h, idx_v, rows_v, sem):
        c = lax.axis_index("c")
        s = lax.axis_index("s")
        wid = s * NSC + c

        def blk(i, carry):
            b = wid + i * (NSC * NT)

            @pl.when(b < NBLKP)
            def _():
                off = b * PB
                pltpu.sync_copy(ql_h.at[pl.ds(off, PB)], idx_v)
                pltpu.async_copy(t_h.at[idx_v], rows_v, sem).wait()
                pltpu.sync_copy(rows_v, out_h.at[pl.ds(off, PB)])

            return carry

        lax.fori_loop(0, (NBLKP + NSC * NT - 1) // (NSC * NT), blk, 0)

    return k(t, qlp)


def _gather_scatter_sc(tab, pair, zextra, v2e):
    if v2e:
        out_type = (jax.ShapeDtypeStruct((NH, D), jnp.float32),
                    jax.ShapeDtypeStruct((NH, CW), jnp.float32))
        stride = NT
        kmax = (NBLK2 + NT - 1) // NT
    else:
        out_type = (jax.ShapeDtypeStruct((NN, D), jnp.float32),
                    jax.ShapeDtypeStruct((NN, D), jnp.float32),
                    jax.ShapeDtypeStruct((NN, CW), jnp.float32),
                    jax.ShapeDtypeStruct((NN, CW), jnp.float32))
        stride = NSC * NT
        kmax = (NBLK2 + NSC * NT - 1) // (NSC * NT)

    @functools.partial(
        pl.kernel,
        out_type=out_type,
        mesh=_MESH,
        scratch_types=[
            pltpu.VMEM((2, 2, PB), jnp.int32),
            pltpu.VMEM((2, SUB, PB), jnp.int32),
            pltpu.VMEM((2, BPB, D), jnp.float32),
            pltpu.VMEM((PB, CW), jnp.float32),
            pltpu.VMEM_SHARED((ACC, D), jnp.float32),
            pltpu.VMEM_SHARED((ACC, CW), jnp.float32),
            pltpu.SemaphoreType.DMA,
            pltpu.SemaphoreType.DMA,
            pltpu.SemaphoreType.DMA,
            pltpu.SemaphoreType.DMA,
            pltpu.SemaphoreType.DMA,
        ],
        compiler_params=pltpu.CompilerParams(use_tc_tiling_on_sc=False),
    )
    def k(tab_h, nid_h, hid_h, zs_h, zc_h, on_h, *rest):
        if v2e:
            sum_h, cnt_h = rest[:2]
            scr = rest[2:]
        else:
            sum0_h, sum1_h, cnt0_h, cnt1_h = rest[:4]
            scr = rest[4:]
        (idx2_v, dst_v, rows_v, ones_v, acc_sh, acc_c,
         isem0, isem1, gsem, ssem0, ssem1) = scr
        c = lax.axis_index("c")
        s = lax.axis_index("s")
        base = c * HALF if v2e else 0
        w0 = s if v2e else s * NSC + c
        gi = 0 if v2e else 1
        isems = (isem0, isem1)
        ssems = (ssem0, ssem1)
        pltpu.sync_copy(on_h, ones_v)
        pltpu.sync_copy(zs_h.at[pl.ds(s * ZSTRIPE, ZSTRIPE)],
                        acc_sh.at[pl.ds(s * ZSTRIPE, ZSTRIPE)])
        pltpu.sync_copy(zc_h.at[pl.ds(s * ZSTRIPE, ZSTRIPE)],
                        acc_c.at[pl.ds(s * ZSTRIPE, ZSTRIPE)])
        plsc.subcore_barrier()
        iota = lax.iota(jnp.int32, 16)
        dummy = HALF + (iota & (NT - 1))

        def drain(p):
            for g in range(SUB):
                pltpu.make_async_copy(rows_v.at[p, pl.ds(g * PB, PB)],
                                      acc_sh.at[dst_v.at[p, g]],
                                      ssems[p]).wait()
                pltpu.make_async_copy(ones_v,
                                      acc_c.at[dst_v.at[p, g]],
                                      ssems[p]).wait()

        def idx_copies(b, p):
            off = b * PB
            return (
                pltpu.make_async_copy(nid_h.at[pl.ds(off, PB)],
                                      idx2_v.at[p, 0], isems[p]),
                pltpu.make_async_copy(hid_h.at[pl.ds(off, PB)],
                                      idx2_v.at[p, 1], isems[p]),
            )

        def prefetch(b, p):
            @pl.when(b < NBLK2)
            def _():
                off = b * PB
                pltpu.async_copy(nid_h.at[pl.ds(off, PB)],
                                 idx2_v.at[p, 0], isems[p])
                pltpu.async_copy(hid_h.at[pl.ds(off, PB)],
                                 idx2_v.at[p, 1], isems[p])

        def substep(kk, p, cp):
            b = w0 + kk * stride

            def do_own(cp):
                @pl.when(cp > 0)
                def _():
                    drain(p)
                for g in range(SUB):
                    for j in range(PB // 16):
                        sl = pl.ds(g * PB + j * 16, 16)
                        sl2 = pl.ds(j * 16, 16)
                        if v2e:
                            loc = idx2_v[p, 1, sl] - base
                            okm = jnp.logical_and(loc >= 0, loc < HALF)
                            dst_v[p, g, sl2] = jnp.where(okm, loc, dummy)
                        else:
                            dst_v[p, g, sl2] = idx2_v[p, 0, sl]
                ds_ = [pltpu.async_copy(tab_h.at[idx2_v.at[p, gi]],
                                        rows_v.at[p, pl.ds(g * PB, PB)], gsem)
                       for g in range(SUB)]
                for d in ds_:
                    d.wait()
                for g in range(SUB):
                    pltpu.async_copy(rows_v.at[p, pl.ds(g * PB, PB)],
                                     acc_sh.at[dst_v.at[p, g]],
                                     ssems[p], add=True)
                    pltpu.async_copy(ones_v,
                                     acc_c.at[dst_v.at[p, g]],
                                     ssems[p], add=True)
                return SUB

            def live(cp):
                for d in idx_copies(b, p):
                    d.wait()
                prefetch(b + stride, p ^ 1)
                if v2e:
                    lo = idx2_v[p, 1, pl.ds(0, 16)][0]
                    hi = idx2_v[p, 1, pl.ds(BPB - 16, 16)][15]
                    own = jnp.logical_and(lo < base + HALF, hi >= base)
                    return lax.cond(own, do_own, lambda cc: cc, cp)
                return do_own(cp)

            return lax.cond(b < NBLK2, live, lambda cc: cc, cp)

        prefetch(w0, 0)

        def outer(k2, carry):
            c0, c1 = carry
            c0 = substep(2 * k2, 0, c0)
            c1 = substep(2 * k2 + 1, 1, c1)
            return (c0, c1)

        c0, c1 = lax.fori_loop(0, (kmax + 1) // 2, outer, (0, 0))

        @pl.when(c0 > 0)
        def _():
            drain(0)

        @pl.when(c1 > 0)
        def _():
            drain(1)

        plsc.subcore_barrier()

        def copy_out(off, n):
            if v2e:
                pltpu.sync_copy(acc_sh.at[pl.ds(off, n)],
                                sum_h.at[pl.ds(base + off, n)])
                pltpu.sync_copy(acc_c.at[pl.ds(off, n)],
                                cnt_h.at[pl.ds(base + off, n)])
            else:
                @pl.when(c == 0)
                def _():
                    pltpu.sync_copy(acc_sh.at[pl.ds(off, n)],
                                    sum0_h.at[pl.ds(off, n)])
                    pltpu.sync_copy(acc_c.at[pl.ds(off, n)],
                                    cnt0_h.at[pl.ds(off, n)])

                @pl.when(c == 1)
                def _():
                    pltpu.sync_copy(acc_sh.at[pl.ds(off, n)],
                                    sum1_h.at[pl.ds(off, n)])
                    pltpu.sync_copy(acc_c.at[pl.ds(off, n)],
                                    cnt1_h.at[pl.ds(off, n)])

        @pl.when(s < NT - 1)
        def _():
            copy_out(s * OSTRIPE, OSTRIPE)

        @pl.when(s == NT - 1)
        def _():
            copy_out((NT - 1) * OSTRIPE, OLAST)

    return k(tab, *pair, *zextra)


def _v2e_sc(tq, pair, zextra):
    return _gather_scatter_sc(tq, pair, zextra, v2e=True)


def _e2v_sc(xe, pair, zextra):
    return _gather_scatter_sc(xe, pair, zextra, v2e=False)


def _mlp_params(p, prefix):
    return [
        p[prefix + '_ln0_g'].reshape(1, D), p[prefix + '_ln0_b'].reshape(1, D),
        p[prefix + '_W1'], p[prefix + '_b1'].reshape(1, D),
        p[prefix + '_ln1_g'].reshape(1, D), p[prefix + '_ln1_b'].reshape(1, D),
        p[prefix + '_W2'], p[prefix + '_b2'].reshape(1, D),
    ]


def kernel(query_labels, node_idx, hyperedge_idx, params):
    p = params
    ql = query_labels.astype(jnp.int32)
    nid = node_idx.astype(jnp.int32)
    hid = hyperedge_idx.astype(jnp.int32)

    zextra = (jnp.zeros((ACC, D), jnp.float32),
              jnp.zeros((ACC, CW), jnp.float32),
              jnp.ones((PB, CW), jnp.float32))
    pair = (nid, hid)

    t = _enc_tc(p['emb'], _mlp_params(p, 'v2e_enc'))
    qlp = jnp.concatenate([ql, jnp.zeros((NNP - NN,), jnp.int32)])
    tq = _perm_sc(t, qlp)
    sum_e, cnt_e = _v2e_sc(tq, pair, zextra)
    xe = _mid_tc(sum_e, cnt_e, _mlp_params(p, 'v2e_dec'), _mlp_params(p, 'e2v_enc'))
    a2 = _e2v_sc(xe, pair, zextra)

    W2p = jnp.zeros((D, D), jnp.float32).at[:, :NCLS].set(p['cls_W2'])
    b2p = jnp.zeros((1, D), jnp.float32).at[:, :NCLS].set(p['cls_b2'])
    ps_cls = [p['cls_W1'], p['cls_b1'].reshape(1, D),
              p['cls_ln_g'].reshape(1, D), p['cls_ln_b'].reshape(1, D),
              W2p, b2p]
    s20, s21, c20, c21 = a2
    y = _final_tc(s20, s21, c20, c21, _mlp_params(p, 'e2v_dec'), ps_cls)
    return y[:, :NCLS]
